# baseline (speedup 1.0000x reference)
import functools

import jax
import jax.numpy as jnp
import numpy as np
from jax import lax
from jax.experimental import pallas as pl
from jax.experimental.pallas import tpu as pltpu
from jax.experimental.pallas import tpu_sc as plsc

PATCH = 4
NUM_PATCHES = 16
MAX_WINDOW = 16
BLOCK = 6
N_EMBED = 16

NUM_WORKERS = 32
NSEG = 2 * BLOCK + 1


def _neighbor_lists(step):
    lists = [[(step, j)] for j in range(NUM_PATCHES)]
    ii = 2
    for c in range(step, -1, -1):
        for j in range(NUM_PATCHES):
            for k in range(-ii + 1, ii):
                for l in range(-ii + 1, ii):
                    if not (j == 0 and l == 0 and ii == 2) and 0 <= j + MAX_WINDOW * k + l < NUM_PATCHES:
                        lists[j].append((c, j + MAX_WINDOW * k + l))
        ii += 1
    c_idx = np.array([c for j in range(NUM_PATCHES) for (c, p) in lists[j]], dtype=np.int64)
    p_idx = np.array([p for j in range(NUM_PATCHES) for (c, p) in lists[j]], dtype=np.int64)
    return c_idx, p_idx


def _base_of(n, c):
    hp, wp = n // 4, n % 4
    return c * 256 + hp * 64 + wp * 4


def _build_static():
    bases = []
    bounds = [0]
    f = np.arange(BLOCK * NUM_PATCHES)
    bases.append(_base_of(f // BLOCK, f % BLOCK))
    row = len(f)
    bounds.append(row)
    for i in range(BLOCK):
        c_idx, p_idx = _neighbor_lists(i)
        ff = c_idx * NUM_PATCHES + p_idx
        b = _base_of(ff // (i + 1), ff % (i + 1))
        for _ in range(2):
            bases.append(b)
            row += len(b)
            bounds.append(row)
    return np.concatenate(bases).astype(np.int32), bounds, row


_BASES, _BOUNDS, _NUM_ROWS = _build_static()
_CHUNK = -(-_NUM_ROWS // NUM_WORKERS)
_LAST_ROWS = _NUM_ROWS - (NUM_WORKERS - 1) * _CHUNK
_XN = BLOCK * 256
_WOFF = _XN
_BOFF = _XN + NSEG * N_EMBED * N_EMBED
_PACKN = _BOFF + NSEG * N_EMBED
_IPAD = _NUM_ROWS + N_EMBED
_BUFN = (_CHUNK + 15) * N_EMBED


def _seg_bound(s, k):
    out = jnp.int32(_BOUNDS[k])
    for j in range(1, NSEG):
        out = jnp.where(s >= j, jnp.int32(_BOUNDS[j + k]), out)
    return out


def _sc_body(pack_hbm, idx_hbm, out_hbm, packv, idxv, buf):
    wid = lax.axis_index("s") * 2 + lax.axis_index("c")
    pltpu.sync_copy(pack_hbm, packv)
    pltpu.sync_copy(idx_hbm, idxv)
    my_lo = wid * _CHUNK
    my_hi = jnp.minimum(my_lo + _CHUNK, _NUM_ROWS)
    lane = lax.broadcasted_iota(jnp.int32, (N_EMBED,), 0)
    offv = (lane >> 2) * 16 + (lane & 3)

    def seg_body(s, carry):
        wbase = _WOFF + s * (N_EMBED * N_EMBED)
        cols = [packv[pl.ds(wbase + N_EMBED * d, N_EMBED)] for d in range(N_EMBED)]
        bias = packv[pl.ds(_BOFF + N_EMBED * s, N_EMBED)]
        lo = jnp.maximum(_seg_bound(s, 0), my_lo)
        hi = jnp.minimum(_seg_bound(s, 1), my_hi)
        ngroups = jnp.maximum(0, (hi - lo + 15) // 16)

        def group(g, c2):
            t0 = lo + 16 * g
            bvec = idxv[pl.ds(t0, N_EMBED)]
            boff = N_EMBED * (t0 - my_lo)
            for r in range(N_EMBED):
                e_row = plsc.load_gather(packv, [bvec[r] + offv])
                acc = bias
                for d in range(N_EMBED):
                    acc = acc + e_row[d] * cols[d]
                buf[pl.ds(boff + N_EMBED * r, N_EMBED)] = acc
            return c2

        lax.fori_loop(0, ngroups, group, 0)
        return carry

    lax.fori_loop(0, NSEG, seg_body, 0)

    @pl.when(wid < NUM_WORKERS - 1)
    def _():
        pltpu.sync_copy(buf.at[pl.ds(0, _CHUNK * N_EMBED)],
                        out_hbm.at[pl.ds(my_lo * N_EMBED, _CHUNK * N_EMBED)])

    @pl.when(wid == NUM_WORKERS - 1)
    def _():
        pltpu.sync_copy(buf.at[pl.ds(0, _LAST_ROWS * N_EMBED)],
                        out_hbm.at[pl.ds(my_lo * N_EMBED, _LAST_ROWS * N_EMBED)])


_sc_call = pl.kernel(
    _sc_body,
    out_type=jax.ShapeDtypeStruct((_NUM_ROWS * N_EMBED,), jnp.float32),
    mesh=plsc.VectorSubcoreMesh(core_axis_name="c", subcore_axis_name="s"),
    compiler_params=pltpu.CompilerParams(needs_layout_passes=False),
    scratch_types=[
        pltpu.VMEM((_PACKN,), jnp.float32),
        pltpu.VMEM((_IPAD,), jnp.int32),
        pltpu.VMEM((_BUFN,), jnp.float32),
    ],
)


@jax.jit
def kernel(x, Wq, Wk, bk, Wv, bv):
    wseg = jnp.stack([Wq] + [m[i] for i in range(BLOCK) for m in (Wk, Wv)])
    wt = jnp.transpose(wseg, (0, 2, 1))
    bseg = jnp.concatenate(
        [jnp.zeros((1, N_EMBED), jnp.float32)]
        + [m[i][None] for i in range(BLOCK) for m in (bk, bv)], axis=0)
    pack = jnp.concatenate([x.reshape(-1), wt.reshape(-1), bseg.reshape(-1)])
    idx = jnp.asarray(np.pad(_BASES, (0, _IPAD - _NUM_ROWS)))
    return _sc_call(pack, idx).reshape(_NUM_ROWS, N_EMBED)


# speedup vs baseline: 1.1658x; 1.1658x over previous
import functools

import jax
import jax.numpy as jnp
import numpy as np
from jax import lax
from jax.experimental import pallas as pl
from jax.experimental.pallas import tpu as pltpu
from jax.experimental.pallas import tpu_sc as plsc

PATCH = 4
NUM_PATCHES = 16
MAX_WINDOW = 16
BLOCK = 6
N_EMBED = 16

NUM_WORKERS = 32
NSEG = 2 * BLOCK + 1


def _neighbor_lists(step):
    lists = [[(step, j)] for j in range(NUM_PATCHES)]
    ii = 2
    for c in range(step, -1, -1):
        for j in range(NUM_PATCHES):
            for k in range(-ii + 1, ii):
                for l in range(-ii + 1, ii):
                    if not (j == 0 and l == 0 and ii == 2) and 0 <= j + MAX_WINDOW * k + l < NUM_PATCHES:
                        lists[j].append((c, j + MAX_WINDOW * k + l))
        ii += 1
    c_idx = np.array([c for j in range(NUM_PATCHES) for (c, p) in lists[j]], dtype=np.int64)
    p_idx = np.array([p for j in range(NUM_PATCHES) for (c, p) in lists[j]], dtype=np.int64)
    return c_idx, p_idx


def _base_of(n, c):
    hp, wp = n // 4, n % 4
    return c * 256 + hp * 64 + wp * 4


def _build_static():
    bases = []
    bounds = [0]
    f = np.arange(BLOCK * NUM_PATCHES)
    bases.append(_base_of(f // BLOCK, f % BLOCK))
    row = len(f)
    bounds.append(row)
    for i in range(BLOCK):
        c_idx, p_idx = _neighbor_lists(i)
        ff = c_idx * NUM_PATCHES + p_idx
        b = _base_of(ff // (i + 1), ff % (i + 1))
        for _ in range(2):
            bases.append(b)
            row += len(b)
            bounds.append(row)
    return np.concatenate(bases).astype(np.int32), bounds, row


_BASES, _BOUNDS, _NUM_ROWS = _build_static()
_CHUNK = 144
_LAST_ROWS = _NUM_ROWS - (NUM_WORKERS - 1) * _CHUNK
_SCAT = -(-_LAST_ROWS // N_EMBED) * N_EMBED
_IPAD = _NUM_ROWS + N_EMBED
_BASES_PADDED = np.pad(_BASES, (0, _IPAD - _NUM_ROWS))


def _kbound(i, k):
    out = jnp.int32(_BOUNDS[k])
    for j in range(1, BLOCK):
        out = jnp.where(i >= j, jnp.int32(_BOUNDS[2 * j + k]), out)
    return out


def _tree_sum(terms):
    while len(terms) > 1:
        nxt = [a + b for a, b in zip(terms[::2], terms[1::2])]
        if len(terms) % 2:
            nxt.append(terms[-1])
        terms = nxt
    return terms[0]


def _sc_body(x_hbm, wq_hbm, wk_hbm, bk_hbm, wv_hbm, bv_hbm, idx_hbm, out_hbm,
             xv, wqv, wkv, bkv, wvv, bvv, idxv, buf, sem):
    wid = lax.axis_index("s") * 2 + lax.axis_index("c")
    cps = [
        pltpu.async_copy(x_hbm, xv, sem),
        pltpu.async_copy(wq_hbm, wqv, sem),
        pltpu.async_copy(wk_hbm.at[pl.ds(0, BLOCK)], wkv, sem),
        pltpu.async_copy(bk_hbm, bkv, sem),
        pltpu.async_copy(wv_hbm.at[pl.ds(0, BLOCK)], wvv, sem),
        pltpu.async_copy(bv_hbm, bvv, sem),
        pltpu.async_copy(idx_hbm, idxv, sem),
    ]
    for cp in cps:
        cp.wait()
    my_lo = wid * _CHUNK
    my_hi = jnp.minimum(my_lo + _CHUNK, _NUM_ROWS)
    lane = lax.broadcasted_iota(jnp.int32, (N_EMBED,), 0)
    zero = lane * 0
    offv = (lane >> 2) * 16 + (lane & 3)

    def run_range(lo, hi, cols, bias):
        ngroups = jnp.maximum(0, (hi - lo + 15) // 16)

        def group(g, c2):
            t0 = lo + 16 * g
            bvec = idxv[pl.ds(t0, N_EMBED)]
            row0 = t0 - my_lo
            for r in range(N_EMBED):
                b = bvec[r] + offv
                e_row = plsc.load_gather(
                    xv, [zero, b >> 8, (b >> 4) & 15, b & 15])
                terms = [e_row[d] * cols[d] for d in range(N_EMBED)]
                if bias is not None:
                    terms.append(bias)
                buf[pl.ds(N_EMBED * (row0 + r), N_EMBED)] = _tree_sum(terms)
            return c2

        lax.fori_loop(0, ngroups, group, 0)

    qcols = [plsc.load_gather(wqv, [lane, zero + d]) for d in range(N_EMBED)]
    run_range(jnp.maximum(0, my_lo), jnp.minimum(_BOUNDS[1], my_hi), qcols, None)

    def step_body(i, carry):
        iv = zero + i
        kcols = [plsc.load_gather(wkv, [iv, lane, zero + d]) for d in range(N_EMBED)]
        run_range(jnp.maximum(_kbound(i, 1), my_lo),
                  jnp.minimum(_kbound(i, 2), my_hi), kcols, bkv[i])
        vcols = [plsc.load_gather(wvv, [iv, lane, zero + d]) for d in range(N_EMBED)]
        run_range(jnp.maximum(_kbound(i, 2), my_lo),
                  jnp.minimum(_kbound(i, 3), my_hi), vcols, bvv[i])
        return carry

    lax.fori_loop(0, BLOCK, step_body, 0)

    @pl.when(wid < NUM_WORKERS - 1)
    def _():
        pltpu.sync_copy(buf.at[pl.ds(0, _CHUNK * N_EMBED)],
                        out_hbm.at[pl.ds(my_lo * N_EMBED, _CHUNK * N_EMBED)])

    @pl.when(wid == NUM_WORKERS - 1)
    def _():
        pltpu.sync_copy(buf.at[pl.ds(0, _LAST_ROWS * N_EMBED)],
                        out_hbm.at[pl.ds(my_lo * N_EMBED, _LAST_ROWS * N_EMBED)])


_sc_call = pl.kernel(
    _sc_body,
    out_type=jax.ShapeDtypeStruct((_NUM_ROWS * N_EMBED,), jnp.float32),
    mesh=plsc.VectorSubcoreMesh(core_axis_name="c", subcore_axis_name="s"),
    compiler_params=pltpu.CompilerParams(needs_layout_passes=False),
    scratch_types=[
        pltpu.VMEM((1, BLOCK, 16, 16), jnp.float32),
        pltpu.VMEM((N_EMBED, N_EMBED), jnp.float32),
        pltpu.VMEM((BLOCK, N_EMBED, N_EMBED), jnp.float32),
        pltpu.VMEM((NUM_PATCHES, N_EMBED), jnp.float32),
        pltpu.VMEM((BLOCK, N_EMBED, N_EMBED), jnp.float32),
        pltpu.VMEM((NUM_PATCHES, N_EMBED), jnp.float32),
        pltpu.VMEM((_IPAD,), jnp.int32),
        pltpu.VMEM(((_CHUNK + 15) * N_EMBED,), jnp.float32),
        pltpu.SemaphoreType.DMA,
    ],
)


@jax.jit
def kernel(x, Wq, Wk, bk, Wv, bv):
    idx = jnp.asarray(_BASES_PADDED)
    return _sc_call(x, Wq, Wk, bk, Wv, bv, idx).reshape(_NUM_ROWS, N_EMBED)
